# baseline (speedup 1.0000x reference)
import functools

import jax
import jax.numpy as jnp
from jax import lax
from jax.experimental import pallas as pl
from jax.experimental.pallas import tpu as pltpu
from jax.experimental.pallas import tpu_sc as plsc

_V = 32000
_B = 16
_P = 50
_R = _B * _P
_H = 1024
_PH = 512
_L2 = 48
_NH = 16
_DH = 64
_NW = 32
_RPB = 64
_RPAD = _B * _RPB
_RPW = _RPAD // _NW
_HALF = _RPW // 2


def _sc_gather(table, idx):
    mesh = plsc.VectorSubcoreMesh(core_axis_name="c", subcore_axis_name="s")

    @functools.partial(
        pl.kernel,
        mesh=mesh,
        out_type=jax.ShapeDtypeStruct((_RPAD, _H), jnp.float32),
        scratch_types=[
            pltpu.VMEM((_RPW,), jnp.int32),
            pltpu.VMEM((_HALF, _H), jnp.float32),
            pltpu.VMEM((_HALF, _H), jnp.float32),
            pltpu.SemaphoreType.DMA,
            pltpu.SemaphoreType.DMA,
        ],
    )
    def k(t_hbm, idx_hbm, out_hbm, idx_v, rows0, rows1, sem0, sem1):
        wid = lax.axis_index("s") * 2 + lax.axis_index("c")
        base = wid * _RPW
        pltpu.sync_copy(idx_hbm.at[pl.ds(base, _RPW)], idx_v)
        cp0 = pltpu.async_copy(t_hbm.at[idx_v.at[pl.ds(0, _HALF)]], rows0, sem0)
        cp1 = pltpu.async_copy(t_hbm.at[idx_v.at[pl.ds(_HALF, _HALF)]], rows1,
                               sem1)
        cp0.wait()
        pltpu.sync_copy(rows0, out_hbm.at[pl.ds(base, _HALF)])
        cp1.wait()
        pltpu.sync_copy(rows1, out_hbm.at[pl.ds(base + _HALF, _HALF)])

    return k(table, idx)


def _mlp(e, w1, b1r, w2, b2r):
    def body(e_ref, w1_ref, b1_ref, w2_ref, b2_ref, out_ref, h_bf):
        i = pl.program_id(0)

        @pl.when(i == 0)
        def _():
            w1 = w1_ref[...].astype(jnp.bfloat16)
            bias1 = b1_ref[...]
            for b in range(_B):
                eb = e_ref[b].astype(jnp.bfloat16)
                acc = lax.dot_general(eb, w1, (((1,), (0,)), ((), ())),
                                      preferred_element_type=jnp.float32)
                h_bf[b] = jnp.tanh(acc + bias1).astype(jnp.bfloat16)

        w = w2_ref[...].astype(jnp.bfloat16)
        bias = b2_ref[0]
        for b in range(_B):
            hb = h_bf[b]
            m = lax.dot_general(hb, w, (((1,), (0,)), ((), ())),
                                preferred_element_type=jnp.float32) + bias
            out_ref[0, pl.ds(b * _P, _P)] = m[:_P].reshape(_P, _NH, _DH)

    return pl.pallas_call(
        body,
        grid=(_L2,),
        in_specs=[
            pl.BlockSpec((_B, _RPB, _H), lambda i: (0, 0, 0)),
            pl.BlockSpec((_H, _PH), lambda i: (0, 0)),
            pl.BlockSpec((1, _PH), lambda i: (0, 0)),
            pl.BlockSpec((_PH, _NH * _DH), lambda i: (0, i)),
            pl.BlockSpec((1, 1, _NH * _DH), lambda i: (i, 0, 0)),
        ],
        out_specs=pl.BlockSpec((1, _R, _NH, _DH),
                               lambda i: (i, 0, 0, 0)),
        out_shape=jax.ShapeDtypeStruct((_L2, _R, _NH, _DH), jnp.float32),
        scratch_shapes=[pltpu.VMEM((_B, _RPB, _PH), jnp.bfloat16)],
        compiler_params=pltpu.CompilerParams(
            dimension_semantics=("arbitrary",)),
    )(e, w1, b1r, w2, b2r)


def kernel(tokens, prompt_table, W1, b1, W2, b2):
    idx = jnp.pad(tokens - _V, ((0, 0), (0, _RPB - _P))).reshape(_RPAD)
    e = _sc_gather(prompt_table, idx)
    out4 = _mlp(e.reshape(_B, _RPB, _H), W1, b1.reshape(1, _PH),
                W2, b2.reshape(_L2, 1, _NH * _DH))
    return out4.reshape(_L2, _B, _P, _NH, _DH).transpose(0, 1, 3, 2, 4)


# speedup vs baseline: 1.0195x; 1.0195x over previous
import functools

import jax
import jax.numpy as jnp
from jax import lax
from jax.experimental import pallas as pl
from jax.experimental.pallas import tpu as pltpu
from jax.experimental.pallas import tpu_sc as plsc

_V = 32000
_B = 16
_P = 50
_R = _B * _P
_NPROMPT = 400
_H = 1024
_PH = 512
_L2 = 48
_NH = 16
_DH = 64
_NW = 32
_RPB = 64
_RPAD = _B * _RPB
_RPW = _RPAD // _NW
_HALF = _RPW // 2


def _mlp1(pt, w1, b1r):
    def body(pt_ref, w1_ref, b1_ref, a_ref):
        p = pt_ref[...].astype(jnp.bfloat16)
        w = w1_ref[...].astype(jnp.bfloat16)
        acc = lax.dot_general(p, w, (((1,), (0,)), ((), ())),
                              preferred_element_type=jnp.float32)
        a_ref[...] = jnp.tanh(acc + b1_ref[...])

    return pl.pallas_call(
        body,
        out_shape=jax.ShapeDtypeStruct((_NPROMPT, _PH), jnp.float32),
    )(pt, w1, b1r)


def _sc_gather(a, idx):
    mesh = plsc.VectorSubcoreMesh(core_axis_name="c", subcore_axis_name="s")

    @functools.partial(
        pl.kernel,
        mesh=mesh,
        out_type=jax.ShapeDtypeStruct((_RPAD, _PH), jnp.float32),
        scratch_types=[
            pltpu.VMEM((_RPW,), jnp.int32),
            pltpu.VMEM((_HALF, _PH), jnp.float32),
            pltpu.VMEM((_HALF, _PH), jnp.float32),
            pltpu.SemaphoreType.DMA,
            pltpu.SemaphoreType.DMA,
        ],
    )
    def k(a_hbm, idx_hbm, out_hbm, idx_v, rows0, rows1, sem0, sem1):
        wid = lax.axis_index("s") * 2 + lax.axis_index("c")
        base = wid * _RPW
        pltpu.sync_copy(idx_hbm.at[pl.ds(base, _RPW)], idx_v)
        cp0 = pltpu.async_copy(a_hbm.at[idx_v.at[pl.ds(0, _HALF)]], rows0, sem0)
        cp1 = pltpu.async_copy(a_hbm.at[idx_v.at[pl.ds(_HALF, _HALF)]], rows1,
                               sem1)
        cp0.wait()
        pltpu.sync_copy(rows0, out_hbm.at[pl.ds(base, _HALF)])
        cp1.wait()
        pltpu.sync_copy(rows1, out_hbm.at[pl.ds(base + _HALF, _HALF)])

    return k(a, idx)


def _mlp2(g, w2, b2r):
    def body(g_ref, w2_ref, b2_ref, out_ref, gbf):
        i = pl.program_id(0)

        @pl.when(i == 0)
        def _():
            gbf[...] = g_ref[...].astype(jnp.bfloat16)

        w = w2_ref[...].astype(jnp.bfloat16)
        bias = b2_ref[0]
        for b in range(_B):
            gb = gbf[b]
            m = lax.dot_general(gb, w, (((1,), (0,)), ((), ())),
                                preferred_element_type=jnp.float32) + bias
            out_ref[0, pl.ds(b * _P, _P)] = m[:_P].reshape(_P, _NH, _DH)

    return pl.pallas_call(
        body,
        grid=(_L2,),
        in_specs=[
            pl.BlockSpec((_B, _RPB, _PH), lambda i: (0, 0, 0)),
            pl.BlockSpec((_PH, _NH * _DH), lambda i: (0, i)),
            pl.BlockSpec((1, 1, _NH * _DH), lambda i: (i, 0, 0)),
        ],
        out_specs=pl.BlockSpec((1, _R, _NH, _DH),
                               lambda i: (i, 0, 0, 0)),
        out_shape=jax.ShapeDtypeStruct((_L2, _R, _NH, _DH), jnp.float32),
        scratch_shapes=[pltpu.VMEM((_B, _RPB, _PH), jnp.bfloat16)],
        compiler_params=pltpu.CompilerParams(
            dimension_semantics=("arbitrary",)),
    )(g, w2, b2r)


def kernel(tokens, prompt_table, W1, b1, W2, b2):
    idx = jnp.pad(tokens - _V, ((0, 0), (0, _RPB - _P))).reshape(_RPAD)
    a = _mlp1(prompt_table, W1, b1.reshape(1, _PH))
    g = _sc_gather(a, idx)
    out4 = _mlp2(g.reshape(_B, _RPB, _PH), W2, b2.reshape(_L2, 1, _NH * _DH))
    return out4.reshape(_L2, _B, _P, _NH, _DH).transpose(0, 1, 3, 2, 4)


# speedup vs baseline: 1.0442x; 1.0242x over previous
import functools

import jax
import jax.numpy as jnp
from jax import lax
from jax.experimental import pallas as pl
from jax.experimental.pallas import tpu as pltpu
from jax.experimental.pallas import tpu_sc as plsc

_V = 32000
_B = 16
_P = 50
_R = _B * _P
_NPROMPT = 400
_H = 1024
_PH = 512
_L2 = 48
_NH = 16
_DH = 64
_NW = 32
_RPB = 64
_RPAD = _B * _RPB
_RPW = _RPAD // _NW
_HALF = _RPW // 2


def _mlp1(pt, w1, b1r):
    def body(pt_ref, w1_ref, b1_ref, a_ref):
        p = pt_ref[...].astype(jnp.bfloat16)
        w = w1_ref[...].astype(jnp.bfloat16)
        acc = lax.dot_general(p, w, (((1,), (0,)), ((), ())),
                              preferred_element_type=jnp.float32)
        a_ref[...] = jnp.tanh(acc + b1_ref[...])

    return pl.pallas_call(
        body,
        out_shape=jax.ShapeDtypeStruct((_NPROMPT, _PH), jnp.float32),
    )(pt, w1, b1r)


def _sc_gather(a, idx):
    mesh = plsc.VectorSubcoreMesh(core_axis_name="c", subcore_axis_name="s")

    @functools.partial(
        pl.kernel,
        mesh=mesh,
        out_type=jax.ShapeDtypeStruct((_RPAD, _PH), jnp.float32),
        scratch_types=[
            pltpu.VMEM((_RPW,), jnp.int32),
            pltpu.VMEM((_HALF, _PH), jnp.float32),
            pltpu.VMEM((_HALF, _PH), jnp.float32),
            pltpu.SemaphoreType.DMA,
            pltpu.SemaphoreType.DMA,
        ],
    )
    def k(a_hbm, idx_hbm, out_hbm, idx_v, rows0, rows1, sem0, sem1):
        wid = lax.axis_index("s") * 2 + lax.axis_index("c")
        base = wid * _RPW
        pltpu.sync_copy(idx_hbm.at[pl.ds(base, _RPW)], idx_v)
        cp0 = pltpu.async_copy(a_hbm.at[idx_v.at[pl.ds(0, _HALF)]], rows0, sem0)
        cp1 = pltpu.async_copy(a_hbm.at[idx_v.at[pl.ds(_HALF, _HALF)]], rows1,
                               sem1)
        cp0.wait()
        pltpu.sync_copy(rows0, out_hbm.at[pl.ds(base, _HALF)])
        cp1.wait()
        pltpu.sync_copy(rows1, out_hbm.at[pl.ds(base + _HALF, _HALF)])

    return k(a, idx)


def _mlp2(g, w2, b2r):
    def body(g_ref, w2_ref, b2_ref, out_ref, gbf):
        i = pl.program_id(0)

        @pl.when(i == 0)
        def _():
            gbf[...] = g_ref[...].astype(jnp.bfloat16)

        w = w2_ref[...].astype(jnp.bfloat16)
        bias = b2_ref[0]
        for half in range(2):
            wh = w[:, half * 1024:(half + 1) * 1024]
            bh = bias[:, half * 1024:(half + 1) * 1024]
            for b in range(_B):
                gb = gbf[b]
                m = lax.dot_general(gb, wh, (((1,), (0,)), ((), ())),
                                    preferred_element_type=jnp.float32) + bh
                out_ref[half, pl.ds(b * _P, _P)] = m[:_P].reshape(_P, _NH, _DH)

    return pl.pallas_call(
        body,
        grid=(_L2 // 2,),
        in_specs=[
            pl.BlockSpec((_B, _RPB, _PH), lambda i: (0, 0, 0)),
            pl.BlockSpec((_PH, 2 * _NH * _DH), lambda i: (0, i)),
            pl.BlockSpec((1, 1, 2 * _NH * _DH), lambda i: (i, 0, 0)),
        ],
        out_specs=pl.BlockSpec((2, _R, _NH, _DH),
                               lambda i: (i, 0, 0, 0)),
        out_shape=jax.ShapeDtypeStruct((_L2, _R, _NH, _DH), jnp.float32),
        scratch_shapes=[pltpu.VMEM((_B, _RPB, _PH), jnp.bfloat16)],
        compiler_params=pltpu.CompilerParams(
            dimension_semantics=("arbitrary",)),
    )(g, w2, b2r)


def kernel(tokens, prompt_table, W1, b1, W2, b2):
    idx = jnp.pad(tokens - _V, ((0, 0), (0, _RPB - _P))).reshape(_RPAD)
    a = _mlp1(prompt_table, W1, b1.reshape(1, _PH))
    g = _sc_gather(a, idx)
    out4 = _mlp2(g.reshape(_B, _RPB, _PH),
                 W2, b2.reshape(_L2 // 2, 1, 2 * _NH * _DH))
    return out4.reshape(_L2, _B, _P, _NH, _DH).transpose(0, 1, 3, 2, 4)


# speedup vs baseline: 1.0479x; 1.0035x over previous
import functools

import jax
import jax.numpy as jnp
from jax import lax
from jax.experimental import pallas as pl
from jax.experimental.pallas import tpu as pltpu
from jax.experimental.pallas import tpu_sc as plsc

_V = 32000
_B = 16
_P = 50
_R = _B * _P
_NPROMPT = 400
_H = 1024
_PH = 512
_L2 = 48
_NH = 16
_DH = 64
_NW = 32
_RPB = 64
_RPAD = _B * _RPB
_RPW = _RPAD // _NW
_HALF = _RPW // 2


def _mlp1(pt, w1, b1r):
    def body(pt_ref, w1_ref, b1_ref, a_ref):
        p = pt_ref[...].astype(jnp.bfloat16)
        w = w1_ref[...].astype(jnp.bfloat16)
        acc = lax.dot_general(p, w, (((1,), (0,)), ((), ())),
                              preferred_element_type=jnp.float32)
        a_ref[...] = jnp.tanh(acc + b1_ref[...])

    return pl.pallas_call(
        body,
        out_shape=jax.ShapeDtypeStruct((_NPROMPT, _PH), jnp.float32),
    )(pt, w1, b1r)


def _sc_gather(a, idx):
    mesh = plsc.VectorSubcoreMesh(core_axis_name="c", subcore_axis_name="s")

    @functools.partial(
        pl.kernel,
        mesh=mesh,
        out_type=jax.ShapeDtypeStruct((_RPAD, _PH), jnp.float32),
        scratch_types=[
            pltpu.VMEM((_RPW,), jnp.int32),
            pltpu.VMEM((_HALF, _PH), jnp.float32),
            pltpu.VMEM((_HALF, _PH), jnp.float32),
            pltpu.SemaphoreType.DMA,
            pltpu.SemaphoreType.DMA,
        ],
    )
    def k(a_hbm, idx_hbm, out_hbm, idx_v, rows0, rows1, sem0, sem1):
        wid = lax.axis_index("s") * 2 + lax.axis_index("c")
        base = wid * _RPW
        pltpu.sync_copy(idx_hbm.at[pl.ds(base, _RPW)], idx_v)
        cp0 = pltpu.async_copy(a_hbm.at[idx_v.at[pl.ds(0, _HALF)]], rows0, sem0)
        cp1 = pltpu.async_copy(a_hbm.at[idx_v.at[pl.ds(_HALF, _HALF)]], rows1,
                               sem1)
        cp0.wait()
        pltpu.sync_copy(rows0, out_hbm.at[pl.ds(base, _HALF)])
        cp1.wait()
        pltpu.sync_copy(rows1, out_hbm.at[pl.ds(base + _HALF, _HALF)])

    return k(a, idx)


def _mlp2(g, w2, b2r):
    def body(g_ref, w2_ref, b2_ref, out_ref, gbf):
        i = pl.program_id(0)

        @pl.when(i == 0)
        def _():
            gbf[...] = g_ref[...].astype(jnp.bfloat16)

        w = w2_ref[...].astype(jnp.bfloat16)
        bias = b2_ref[0]
        for half in range(3):
            wh = w[:, half * 1024:(half + 1) * 1024]
            bh = bias[:, half * 1024:(half + 1) * 1024]
            for b in range(_B):
                gb = gbf[b]
                m = lax.dot_general(gb, wh, (((1,), (0,)), ((), ())),
                                    preferred_element_type=jnp.float32) + bh
                out_ref[half, pl.ds(b * _P, _P)] = m[:_P].reshape(_P, _NH, _DH)

    return pl.pallas_call(
        body,
        grid=(_L2 // 3,),
        in_specs=[
            pl.BlockSpec((_B, _RPB, _PH), lambda i: (0, 0, 0)),
            pl.BlockSpec((_PH, 3 * _NH * _DH), lambda i: (0, i)),
            pl.BlockSpec((1, 1, 3 * _NH * _DH), lambda i: (i, 0, 0)),
        ],
        out_specs=pl.BlockSpec((3, _R, _NH, _DH),
                               lambda i: (i, 0, 0, 0)),
        out_shape=jax.ShapeDtypeStruct((_L2, _R, _NH, _DH), jnp.float32),
        scratch_shapes=[pltpu.VMEM((_B, _RPB, _PH), jnp.bfloat16)],
        compiler_params=pltpu.CompilerParams(
            dimension_semantics=("arbitrary",)),
    )(g, w2, b2r)


def kernel(tokens, prompt_table, W1, b1, W2, b2):
    idx = jnp.pad(tokens - _V, ((0, 0), (0, _RPB - _P))).reshape(_RPAD)
    a = _mlp1(prompt_table, W1, b1.reshape(1, _PH))
    g = _sc_gather(a, idx)
    out4 = _mlp2(g.reshape(_B, _RPB, _PH),
                 W2, b2.reshape(_L2 // 3, 1, 3 * _NH * _DH))
    return out4.reshape(_L2, _B, _P, _NH, _DH).transpose(0, 1, 3, 2, 4)


# speedup vs baseline: 1.0535x; 1.0054x over previous
import functools

import jax
import jax.numpy as jnp
from jax import lax
from jax.experimental import pallas as pl
from jax.experimental.pallas import tpu as pltpu
from jax.experimental.pallas import tpu_sc as plsc

_V = 32000
_B = 16
_P = 50
_R = _B * _P
_NPROMPT = 400
_H = 1024
_PH = 512
_L2 = 48
_NH = 16
_DH = 64
_NW = 32
_RPB = 64
_RPAD = _B * _RPB
_RPW = _RPAD // 16
_HALF = _RPW // 2


def _mlp1(pt, w1, b1r):
    def body(pt_ref, w1_ref, b1_ref, a_ref):
        p = pt_ref[...].astype(jnp.bfloat16)
        w = w1_ref[...].astype(jnp.bfloat16)
        acc = lax.dot_general(p, w, (((1,), (0,)), ((), ())),
                              preferred_element_type=jnp.float32)
        a_ref[...] = jnp.tanh(acc + b1_ref[...])

    return pl.pallas_call(
        body,
        out_shape=jax.ShapeDtypeStruct((_NPROMPT, _PH), jnp.float32),
    )(pt, w1, b1r)


def _sc_gather(a, idx):
    mesh = plsc.VectorSubcoreMesh(core_axis_name="c", subcore_axis_name="s", num_cores=1)

    @functools.partial(
        pl.kernel,
        mesh=mesh,
        out_type=jax.ShapeDtypeStruct((_RPAD, _PH), jnp.float32),
        scratch_types=[
            pltpu.VMEM((_RPW,), jnp.int32),
            pltpu.VMEM((_HALF, _PH), jnp.float32),
            pltpu.VMEM((_HALF, _PH), jnp.float32),
            pltpu.SemaphoreType.DMA,
            pltpu.SemaphoreType.DMA,
        ],
    )
    def k(a_hbm, idx_hbm, out_hbm, idx_v, rows0, rows1, sem0, sem1):
        wid = lax.axis_index("s") + lax.axis_index("c")
        base = wid * _RPW
        pltpu.sync_copy(idx_hbm.at[pl.ds(base, _RPW)], idx_v)
        cp0 = pltpu.async_copy(a_hbm.at[idx_v.at[pl.ds(0, _HALF)]], rows0, sem0)
        cp1 = pltpu.async_copy(a_hbm.at[idx_v.at[pl.ds(_HALF, _HALF)]], rows1,
                               sem1)
        cp0.wait()
        pltpu.sync_copy(rows0, out_hbm.at[pl.ds(base, _HALF)])
        cp1.wait()
        pltpu.sync_copy(rows1, out_hbm.at[pl.ds(base + _HALF, _HALF)])

    return k(a, idx)


def _mlp2(g, w2, b2r):
    def body(g_ref, w2_ref, b2_ref, out_ref, gbf):
        i = pl.program_id(0)

        @pl.when(i == 0)
        def _():
            gbf[...] = g_ref[...].astype(jnp.bfloat16)

        w = w2_ref[...].astype(jnp.bfloat16)
        bias = b2_ref[0]
        for half in range(3):
            wh = w[:, half * 1024:(half + 1) * 1024]
            bh = bias[:, half * 1024:(half + 1) * 1024]
            for b in range(_B):
                gb = gbf[b]
                m = lax.dot_general(gb, wh, (((1,), (0,)), ((), ())),
                                    preferred_element_type=jnp.float32) + bh
                out_ref[half, pl.ds(b * _P, _P)] = m[:_P].reshape(_P, _NH, _DH)

    return pl.pallas_call(
        body,
        grid=(_L2 // 3,),
        in_specs=[
            pl.BlockSpec((_B, _RPB, _PH), lambda i: (0, 0, 0)),
            pl.BlockSpec((_PH, 3 * _NH * _DH), lambda i: (0, i)),
            pl.BlockSpec((1, 1, 3 * _NH * _DH), lambda i: (i, 0, 0)),
        ],
        out_specs=pl.BlockSpec((3, _R, _NH, _DH),
                               lambda i: (i, 0, 0, 0)),
        out_shape=jax.ShapeDtypeStruct((_L2, _R, _NH, _DH), jnp.float32),
        scratch_shapes=[pltpu.VMEM((_B, _RPB, _PH), jnp.bfloat16)],
        compiler_params=pltpu.CompilerParams(
            dimension_semantics=("arbitrary",)),
    )(g, w2, b2r)


def kernel(tokens, prompt_table, W1, b1, W2, b2):
    idx = jnp.pad(tokens - _V, ((0, 0), (0, _RPB - _P))).reshape(_RPAD)
    a = _mlp1(prompt_table, W1, b1.reshape(1, _PH))
    g = _sc_gather(a, idx)
    out4 = _mlp2(g.reshape(_B, _RPB, _PH),
                 W2, b2.reshape(_L2 // 3, 1, 3 * _NH * _DH))
    return out4.reshape(_L2, _B, _P, _NH, _DH).transpose(0, 1, 3, 2, 4)


# speedup vs baseline: 1.0535x; 1.0000x over previous
import functools

import jax
import jax.numpy as jnp
from jax import lax
from jax.experimental import pallas as pl
from jax.experimental.pallas import tpu as pltpu
from jax.experimental.pallas import tpu_sc as plsc

_V = 32000
_B = 16
_P = 50
_R = _B * _P
_NPROMPT = 400
_H = 1024
_PH = 512
_L2 = 48
_NH = 16
_DH = 64
_RPB = 64
_RPAD = _B * _RPB
_RPW = _RPAD // 16
_HALF = _RPW // 2


def _mlp1(pt, w1, b1r):
    def body(pt_ref, w1_ref, b1_ref, a_ref):
        p = pt_ref[...].astype(jnp.bfloat16)
        w = w1_ref[...].astype(jnp.bfloat16)
        acc = lax.dot_general(p, w, (((1,), (0,)), ((), ())),
                              preferred_element_type=jnp.float32)
        a_ref[...] = jnp.tanh(acc + b1_ref[...])

    return pl.pallas_call(
        body,
        out_shape=jax.ShapeDtypeStruct((_NPROMPT, _PH), jnp.float32),
    )(pt, w1, b1r)


def _sc_gather(a, idx):
    mesh = plsc.VectorSubcoreMesh(core_axis_name="c", subcore_axis_name="s",
                                  num_cores=1)

    @functools.partial(
        pl.kernel,
        mesh=mesh,
        out_type=jax.ShapeDtypeStruct((_RPAD, _PH), jnp.float32),
        scratch_types=[
            pltpu.VMEM((_RPW,), jnp.int32),
            pltpu.VMEM((_HALF, _PH), jnp.float32),
            pltpu.VMEM((_HALF, _PH), jnp.float32),
            pltpu.SemaphoreType.DMA,
            pltpu.SemaphoreType.DMA,
        ],
    )
    def k(a_hbm, idx_hbm, out_hbm, idx_v, rows0, rows1, sem0, sem1):
        wid = lax.axis_index("s") + lax.axis_index("c")
        base = wid * _RPW
        pltpu.sync_copy(idx_hbm.at[pl.ds(base, _RPW)], idx_v)
        cp0 = pltpu.async_copy(a_hbm.at[idx_v.at[pl.ds(0, _HALF)]], rows0, sem0)
        cp1 = pltpu.async_copy(a_hbm.at[idx_v.at[pl.ds(_HALF, _HALF)]], rows1,
                               sem1)
        cp0.wait()
        pltpu.sync_copy(rows0, out_hbm.at[pl.ds(base, _HALF)])
        cp1.wait()
        pltpu.sync_copy(rows1, out_hbm.at[pl.ds(base + _HALF, _HALF)])

    return k(a, idx)


def _mlp2(g, w2, b2r):
    def body(g_ref, w2_ref, b2_ref, out_ref, gbf):
        i = pl.program_id(0)

        @pl.when(i == 0)
        def _():
            gbf[...] = g_ref[...].astype(jnp.bfloat16)

        w = w2_ref[...].astype(jnp.bfloat16)
        bias = b2_ref[0]
        for half in range(3):
            wh = w[:, half * 1024:(half + 1) * 1024]
            bh = bias[:, half * 1024:(half + 1) * 1024]
            for b in range(_B):
                gb = gbf[b]
                m = lax.dot_general(gb, wh, (((1,), (0,)), ((), ())),
                                    preferred_element_type=jnp.float32) + bh
                out_ref[half, pl.ds(b * _P, _P)] = m[:_P].reshape(_P, _NH, _DH)

    return pl.pallas_call(
        body,
        grid=(_L2 // 3,),
        in_specs=[
            pl.BlockSpec((_B, _RPB, _PH), lambda i: (0, 0, 0)),
            pl.BlockSpec((_PH, 3 * _NH * _DH), lambda i: (0, i)),
            pl.BlockSpec((1, 1, 3 * _NH * _DH), lambda i: (i, 0, 0)),
        ],
        out_specs=pl.BlockSpec((3, _R, _NH, _DH),
                               lambda i: (i, 0, 0, 0)),
        out_shape=jax.ShapeDtypeStruct((_L2, _R, _NH, _DH), jnp.float32),
        scratch_shapes=[pltpu.VMEM((_B, _RPB, _PH), jnp.bfloat16)],
        compiler_params=pltpu.CompilerParams(
            dimension_semantics=("arbitrary",)),
    )(g, w2, b2r)


def kernel(tokens, prompt_table, W1, b1, W2, b2):
    idx = jnp.pad(tokens - _V, ((0, 0), (0, _RPB - _P))).reshape(_RPAD)
    a = _mlp1(prompt_table, W1, b1.reshape(1, _PH))
    g = _sc_gather(a, idx)
    out4 = _mlp2(g.reshape(_B, _RPB, _PH),
                 W2, b2.reshape(_L2 // 3, 1, 3 * _NH * _DH))
    return out4.reshape(_L2, _B, _P, _NH, _DH).transpose(0, 1, 3, 2, 4)
